# baseline (speedup 1.0000x reference)
import functools

import jax
import jax.numpy as jnp
from jax import lax
from jax.experimental import pallas as pl
from jax.experimental.pallas import tpu as pltpu
from jax.experimental.pallas import tpu_sc as plsc

_K = 100000
_D = 768
_S = 2048
_SBLK = 256

_NW = 32
_CH = 40
_NCH = 32
_RPW = _CH * _NCH
_KSC = _NW * _RPW
_KTC = _K - _KSC
_KBLK = 4920
_NJ = _D // 16

_BIG = 2**30
_ER = 800

_DNUMS = lax.GatherDimensionNumbers(
    offset_dims=(), collapsed_slice_dims=(0,), start_index_map=(0,))


def _lane_shuffle(s, perm16):
    return lax.gather(s, perm16[:, None], _DNUMS, (1,),
                      mode=lax.GatherScatterMode.PROMISE_IN_BOUNDS)


def _sc_scan_body(keys_hbm, q_hbm, d2s_hbm, idxs_hbm,
                  q_v, buf0, buf1, bd_v, bi_v, sem0, sem1):
    wid = lax.axis_index("s") * 2 + lax.axis_index("c")
    row0 = _KTC + wid * _RPW

    pltpu.sync_copy(q_hbm, q_v)
    pltpu.make_async_copy(keys_hbm.at[pl.ds(row0, _CH)], buf0, sem0).start()
    pltpu.make_async_copy(
        keys_hbm.at[pl.ds(row0 + _CH, _CH)], buf1, sem1).start()

    lanes = lax.iota(jnp.int32, 16)
    perms = [lanes ^ k for k in (1, 2, 4, 8)]
    ones = jnp.ones((16,), jnp.int32)

    def compute_chunk(buf, carry):
        def rowgrp(g, cr):
            bdv, biv, rv = cr
            base = g * 8
            accs = [jnp.zeros((16,), jnp.float32) for _ in range(8)]
            for j in range(_NJ):
                qj = q_v[pl.ds(j * 16, 16)]
                for r8 in range(8):
                    t = buf[base + r8, pl.ds(j * 16, 16)]
                    d = t - qj
                    accs[r8] = accs[r8] + d * d
            for r8 in range(8):
                s = accs[r8]
                for p in perms:
                    s = s + _lane_shuffle(s, p)
                better = s < bdv
                bdv = jnp.where(better, s, bdv)
                biv = jnp.where(better, rv, biv)
                rv = rv + ones
            return bdv, biv, rv
        return lax.fori_loop(0, _CH // 8, rowgrp, carry)

    def step(c, buf, sem, carry):
        pltpu.make_async_copy(keys_hbm.at[pl.ds(row0, _CH)], buf, sem).wait()
        carry = compute_chunk(buf, carry)

        @pl.when(c + 2 < _NCH)
        def _prefetch():
            pltpu.make_async_copy(
                keys_hbm.at[pl.ds(row0 + (c + 2) * _CH, _CH)], buf,
                sem).start()

        return carry

    def pair(p, carry):
        carry = step(2 * p, buf0, sem0, carry)
        carry = step(2 * p + 1, buf1, sem1, carry)
        return carry

    bdv, biv, _ = lax.fori_loop(
        0, _NCH // 2, pair,
        (jnp.full((16,), jnp.inf, jnp.float32),
         jnp.zeros((16,), jnp.int32),
         jnp.zeros((16,), jnp.int32)))

    for t in range(8):
        bd_v[t, :] = bdv
        bi_v[t, :] = biv
    pltpu.sync_copy(bd_v, d2s_hbm.at[pl.ds(wid * 8, 8)])
    pltpu.sync_copy(bi_v, idxs_hbm.at[pl.ds(wid * 8, 8)])


def _sc_scan(keys, q1d):
    mesh = plsc.VectorSubcoreMesh(core_axis_name="c", subcore_axis_name="s")
    fn = pl.kernel(
        _sc_scan_body,
        out_type=[
            jax.ShapeDtypeStruct((_NW * 8, 16), jnp.float32),
            jax.ShapeDtypeStruct((_NW * 8, 16), jnp.int32),
        ],
        mesh=mesh,
        scratch_types=[
            pltpu.VMEM((_D,), jnp.float32),
            pltpu.VMEM((_CH, _D), jnp.float32),
            pltpu.VMEM((_CH, _D), jnp.float32),
            pltpu.VMEM((8, 16), jnp.float32),
            pltpu.VMEM((8, 16), jnp.int32),
            pltpu.SemaphoreType.DMA,
            pltpu.SemaphoreType.DMA,
        ],
    )
    return fn(keys, q1d)


def _scan_body(q_ref, keys_ref, d2_out, idx_out, bestd_ref, besti_ref):
    step = pl.program_id(0)

    @pl.when(step == 0)
    def _init():
        bestd_ref[0] = jnp.float32(jnp.inf)
        besti_ref[0] = jnp.int32(0)

    q = q_ref[...]
    k = keys_ref[...]
    diff = k - q
    d2 = jnp.sum(diff * diff, axis=1, keepdims=True)
    minv = jnp.min(d2)
    iota = lax.broadcasted_iota(jnp.int32, (_KBLK, 1), 0)
    lidx = jnp.min(jnp.where(d2 == minv, iota, _BIG))

    better = minv < bestd_ref[0]
    bestd_ref[0] = jnp.where(better, minv, bestd_ref[0])
    besti_ref[0] = jnp.where(better, step * _KBLK + lidx, besti_ref[0])

    @pl.when(step == pl.num_programs(0) - 1)
    def _fin():
        d2_out[0, 0] = bestd_ref[0]
        idx_out[0, 0] = besti_ref[0]


def _tc_scan(query_2d, keys):
    grid = (_KTC // _KBLK,)
    return pl.pallas_call(
        _scan_body,
        grid=grid,
        in_specs=[
            pl.BlockSpec((1, _D), lambda i: (0, 0)),
            pl.BlockSpec((_KBLK, _D), lambda i: (i, 0)),
        ],
        out_specs=[
            pl.BlockSpec(memory_space=pltpu.SMEM),
            pl.BlockSpec(memory_space=pltpu.SMEM),
        ],
        out_shape=[
            jax.ShapeDtypeStruct((1, 1), jnp.float32),
            jax.ShapeDtypeStruct((1, 1), jnp.int32),
        ],
        scratch_shapes=[
            pltpu.SMEM((1,), jnp.float32),
            pltpu.SMEM((1,), jnp.int32),
        ],
    )(query_2d, keys)


def _fused_body(d2_ref, idx_ref, tok_ref, d2sc_ref, idxsc_ref,
                x_ref, w_ref, b_ref, values_ref, eps_ref, out_ref,
                val_ref, epsv_ref, pick_ref, lane_ref, sem, sem2):
    i = pl.program_id(0)

    @pl.when(i == 0)
    def _merge_and_fetch():
        d2sc = d2sc_ref[...]
        rows8 = lax.broadcasted_iota(jnp.int32, (_NW * 8, 16), 0)
        gidx = _KTC + (rows8 // 8) * _RPW + idxsc_ref[...]
        msc = jnp.min(d2sc)
        isc = jnp.min(jnp.where(d2sc == msc, gidx, _BIG))
        use_sc = msc < d2_ref[0, 0]
        d2min = jnp.where(use_sc, msc, d2_ref[0, 0])
        idx = jnp.where(use_sc, isc, idx_ref[0, 0])
        pick_ref[0] = d2min
        lane_ref[0] = idx % 128
        copy = pltpu.make_async_copy(
            values_ref.at[pl.ds(idx, 1)], val_ref, sem)
        copy.start()
        copy2 = pltpu.make_async_copy(
            eps_ref.at[pl.ds(idx // 128, 1)], epsv_ref, sem2)
        copy2.start()
        copy.wait()
        copy2.wait()

    out = lax.dot_general(
        x_ref[...], w_ref[...], (((1,), (1,)), ((), ())),
        preferred_element_type=jnp.float32)
    out = out + b_ref[...]

    laneio = lax.broadcasted_iota(jnp.int32, (1, 128), 1)
    eps = jnp.sum(jnp.where(laneio == lane_ref[0], epsv_ref[...],
                            jnp.float32(0.0)))
    cond = (eps >= 0.0) & (pick_ref[0] <= eps * eps)
    rows = i * _SBLK + lax.broadcasted_iota(jnp.int32, (_SBLK, 1), 0)
    mask = (rows < tok_ref[0, 0]) & cond
    out_ref[...] = jnp.where(mask, val_ref[...], out)


def _fused_out(x2d, W, b2d, values, epsilons, d2, idx, tok, d2sc, idxsc):
    grid = (_S // _SBLK,)
    return pl.pallas_call(
        _fused_body,
        grid=grid,
        in_specs=[
            pl.BlockSpec(memory_space=pltpu.SMEM),
            pl.BlockSpec(memory_space=pltpu.SMEM),
            pl.BlockSpec(memory_space=pltpu.SMEM),
            pl.BlockSpec((_NW * 8, 16), lambda i: (0, 0)),
            pl.BlockSpec((_NW * 8, 16), lambda i: (0, 0)),
            pl.BlockSpec((_SBLK, _D), lambda i: (i, 0)),
            pl.BlockSpec((_D, _D), lambda i: (0, 0)),
            pl.BlockSpec((1, _D), lambda i: (0, 0)),
            pl.BlockSpec(memory_space=pl.ANY),
            pl.BlockSpec(memory_space=pl.ANY),
        ],
        out_specs=pl.BlockSpec((_SBLK, _D), lambda i: (i, 0)),
        out_shape=jax.ShapeDtypeStruct((_S, _D), jnp.float32),
        scratch_shapes=[
            pltpu.VMEM((1, _D), jnp.float32),
            pltpu.VMEM((1, 128), jnp.float32),
            pltpu.SMEM((1,), jnp.float32),
            pltpu.SMEM((1,), jnp.int32),
            pltpu.SemaphoreType.DMA,
            pltpu.SemaphoreType.DMA,
        ],
    )(d2, idx, tok, d2sc, idxsc, x2d, W, b2d, values, epsilons)


def kernel(x, W, b, keys, values, epsilons, key_id):
    tok = jnp.minimum(jnp.asarray(key_id, jnp.int32), x.shape[1] - 1)
    x2d = x[0]
    query = lax.dynamic_slice_in_dim(x2d, tok, 1, axis=0)
    eps2d = jnp.pad(epsilons.reshape(_K), (0, _ER * 128 - _K)).reshape(
        _ER, 128)
    d2sc, idxsc = _sc_scan(keys, query.reshape(_D))
    d2, idx = _tc_scan(query, keys)
    out = _fused_out(x2d, W, b.reshape(1, _D), values, eps2d,
                     d2, idx, tok.reshape(1, 1), d2sc, idxsc)
    return out[None]


# speedup vs baseline: 1.0269x; 1.0269x over previous
import functools

import jax
import jax.numpy as jnp
from jax import lax
from jax.experimental import pallas as pl
from jax.experimental.pallas import tpu as pltpu
from jax.experimental.pallas import tpu_sc as plsc

_K = 100000
_D = 768
_S = 2048
_SBLK = 256

_NW = 32
_CH = 72
_NCH = 20
_RPW = _CH * _NCH
_KSC = _NW * _RPW
_KTC = _K - _KSC
_KBLK = 5392
_NJ = _D // 16

_BIG = 2**30
_ER = 800

_DNUMS = lax.GatherDimensionNumbers(
    offset_dims=(), collapsed_slice_dims=(0,), start_index_map=(0,))


def _lane_shuffle(s, perm16):
    return lax.gather(s, perm16[:, None], _DNUMS, (1,),
                      mode=lax.GatherScatterMode.PROMISE_IN_BOUNDS)


def _sc_scan_body(keys_hbm, q_hbm, d2s_hbm, idxs_hbm,
                  q_v, buf0, buf1, bd_v, bi_v, sem0, sem1):
    wid = lax.axis_index("s") * 2 + lax.axis_index("c")
    row0 = _KTC + wid * _RPW

    pltpu.sync_copy(q_hbm, q_v)
    pltpu.make_async_copy(keys_hbm.at[pl.ds(row0, _CH)], buf0, sem0).start()
    pltpu.make_async_copy(
        keys_hbm.at[pl.ds(row0 + _CH, _CH)], buf1, sem1).start()

    lanes = lax.iota(jnp.int32, 16)
    perms = [lanes ^ k for k in (1, 2, 4, 8)]
    ones = jnp.ones((16,), jnp.int32)

    def compute_chunk(buf, carry):
        def rowgrp(g, cr):
            bdv, biv, rv = cr
            base = g * 8
            accs = [jnp.zeros((16,), jnp.float32) for _ in range(8)]
            for j in range(_NJ):
                qj = q_v[pl.ds(j * 16, 16)]
                for r8 in range(8):
                    t = buf[base + r8, pl.ds(j * 16, 16)]
                    d = t - qj
                    accs[r8] = accs[r8] + d * d
            for r8 in range(8):
                s = accs[r8]
                for p in perms:
                    s = s + _lane_shuffle(s, p)
                better = s < bdv
                bdv = jnp.where(better, s, bdv)
                biv = jnp.where(better, rv, biv)
                rv = rv + ones
            return bdv, biv, rv
        return lax.fori_loop(0, _CH // 8, rowgrp, carry)

    def step(c, buf, sem, carry):
        pltpu.make_async_copy(keys_hbm.at[pl.ds(row0, _CH)], buf, sem).wait()
        carry = compute_chunk(buf, carry)

        @pl.when(c + 2 < _NCH)
        def _prefetch():
            pltpu.make_async_copy(
                keys_hbm.at[pl.ds(row0 + (c + 2) * _CH, _CH)], buf,
                sem).start()

        return carry

    def pair(p, carry):
        carry = step(2 * p, buf0, sem0, carry)
        carry = step(2 * p + 1, buf1, sem1, carry)
        return carry

    bdv, biv, _ = lax.fori_loop(
        0, _NCH // 2, pair,
        (jnp.full((16,), jnp.inf, jnp.float32),
         jnp.zeros((16,), jnp.int32),
         jnp.zeros((16,), jnp.int32)))

    for t in range(8):
        bd_v[t, :] = bdv
        bi_v[t, :] = biv
    pltpu.sync_copy(bd_v, d2s_hbm.at[pl.ds(wid * 8, 8)])
    pltpu.sync_copy(bi_v, idxs_hbm.at[pl.ds(wid * 8, 8)])


def _sc_scan(keys, q1d):
    mesh = plsc.VectorSubcoreMesh(core_axis_name="c", subcore_axis_name="s")
    fn = pl.kernel(
        _sc_scan_body,
        out_type=[
            jax.ShapeDtypeStruct((_NW * 8, 16), jnp.float32),
            jax.ShapeDtypeStruct((_NW * 8, 16), jnp.int32),
        ],
        mesh=mesh,
        scratch_types=[
            pltpu.VMEM((_D,), jnp.float32),
            pltpu.VMEM((_CH, _D), jnp.float32),
            pltpu.VMEM((_CH, _D), jnp.float32),
            pltpu.VMEM((8, 16), jnp.float32),
            pltpu.VMEM((8, 16), jnp.int32),
            pltpu.SemaphoreType.DMA,
            pltpu.SemaphoreType.DMA,
        ],
    )
    return fn(keys, q1d)


def _scan_body(q_ref, keys_ref, x_ref, w_ref, b_ref,
               d2_out, idx_out, y_out, bestd_ref, besti_ref):
    step = pl.program_id(0)

    @pl.when(step == 0)
    def _init():
        bestd_ref[0] = jnp.float32(jnp.inf)
        besti_ref[0] = jnp.int32(0)

    @pl.when(step < _S // _SBLK)
    def _mm():
        y = lax.dot_general(
            x_ref[...], w_ref[...], (((1,), (1,)), ((), ())),
            preferred_element_type=jnp.float32)
        y_out[...] = y + b_ref[...]

    q = q_ref[...]
    k = keys_ref[...]
    diff = k - q
    d2 = jnp.sum(diff * diff, axis=1, keepdims=True)
    minv = jnp.min(d2)
    iota = lax.broadcasted_iota(jnp.int32, (_KBLK, 1), 0)
    lidx = jnp.min(jnp.where(d2 == minv, iota, _BIG))

    better = minv < bestd_ref[0]
    bestd_ref[0] = jnp.where(better, minv, bestd_ref[0])
    besti_ref[0] = jnp.where(better, step * _KBLK + lidx, besti_ref[0])

    @pl.when(step == pl.num_programs(0) - 1)
    def _fin():
        d2_out[0, 0] = bestd_ref[0]
        idx_out[0, 0] = besti_ref[0]


def _tc_scan_mm(query_2d, keys, x2d, W, b2d):
    grid = (_KTC // _KBLK,)
    nxb = _S // _SBLK
    return pl.pallas_call(
        _scan_body,
        grid=grid,
        in_specs=[
            pl.BlockSpec((1, _D), lambda i: (0, 0)),
            pl.BlockSpec((_KBLK, _D), lambda i: (i, 0)),
            pl.BlockSpec((_SBLK, _D), lambda i: (jnp.minimum(i, nxb - 1), 0)),
            pl.BlockSpec((_D, _D), lambda i: (0, 0)),
            pl.BlockSpec((1, _D), lambda i: (0, 0)),
        ],
        out_specs=[
            pl.BlockSpec(memory_space=pltpu.SMEM),
            pl.BlockSpec(memory_space=pltpu.SMEM),
            pl.BlockSpec((_SBLK, _D), lambda i: (jnp.minimum(i, nxb - 1), 0)),
        ],
        out_shape=[
            jax.ShapeDtypeStruct((1, 1), jnp.float32),
            jax.ShapeDtypeStruct((1, 1), jnp.int32),
            jax.ShapeDtypeStruct((_S, _D), jnp.float32),
        ],
        scratch_shapes=[
            pltpu.SMEM((1,), jnp.float32),
            pltpu.SMEM((1,), jnp.int32),
        ],
    )(query_2d, keys, x2d, W, b2d)


def _final_body(d2_ref, idx_ref, tok_ref, d2sc_ref, idxsc_ref,
                values_ref, eps_ref, y_ref, out_ref,
                val_ref, epsv_ref, ybuf_ref, sem, sem2):
    d2sc = d2sc_ref[...]
    rows8 = lax.broadcasted_iota(jnp.int32, (_NW * 8, 16), 0)
    gidx = _KTC + (rows8 // 8) * _RPW + idxsc_ref[...]
    msc = jnp.min(d2sc)
    isc = jnp.min(jnp.where(d2sc == msc, gidx, _BIG))
    use_sc = msc < d2_ref[0, 0]
    d2min = jnp.where(use_sc, msc, d2_ref[0, 0])
    idx = jnp.where(use_sc, isc, idx_ref[0, 0])

    copy = pltpu.make_async_copy(values_ref.at[pl.ds(idx, 1)], val_ref, sem)
    copy.start()
    copy2 = pltpu.make_async_copy(
        eps_ref.at[pl.ds(idx // 128, 1)], epsv_ref, sem2)
    copy2.start()
    copy.wait()
    copy2.wait()

    laneio = lax.broadcasted_iota(jnp.int32, (1, 128), 1)
    eps = jnp.sum(jnp.where(laneio == idx % 128, epsv_ref[...],
                            jnp.float32(0.0)))
    cond = (eps >= 0.0) & (d2min <= eps * eps)
    tok = tok_ref[0, 0]

    for b in range(_S // _SBLK):
        @pl.when(cond & (b * _SBLK < tok))
        def _rewrite(b=b):
            ld = pltpu.make_async_copy(
                y_ref.at[pl.ds(b * _SBLK, _SBLK)], ybuf_ref, sem)
            ld.start()
            ld.wait()
            rows = b * _SBLK + lax.broadcasted_iota(
                jnp.int32, (_SBLK, 1), 0)
            ybuf_ref[...] = jnp.where(rows < tok, val_ref[...],
                                      ybuf_ref[...])
            st = pltpu.make_async_copy(
                ybuf_ref, out_ref.at[pl.ds(b * _SBLK, _SBLK)], sem)
            st.start()
            st.wait()


def _finalize(y, values, eps2d, d2, idx, tok, d2sc, idxsc):
    return pl.pallas_call(
        _final_body,
        in_specs=[
            pl.BlockSpec(memory_space=pltpu.SMEM),
            pl.BlockSpec(memory_space=pltpu.SMEM),
            pl.BlockSpec(memory_space=pltpu.SMEM),
            pl.BlockSpec((_NW * 8, 16), lambda: (0, 0)),
            pl.BlockSpec((_NW * 8, 16), lambda: (0, 0)),
            pl.BlockSpec(memory_space=pl.ANY),
            pl.BlockSpec(memory_space=pl.ANY),
            pl.BlockSpec(memory_space=pl.ANY),
        ],
        out_specs=pl.BlockSpec(memory_space=pl.ANY),
        out_shape=jax.ShapeDtypeStruct((_S, _D), jnp.float32),
        input_output_aliases={7: 0},
        scratch_shapes=[
            pltpu.VMEM((1, _D), jnp.float32),
            pltpu.VMEM((1, 128), jnp.float32),
            pltpu.VMEM((_SBLK, _D), jnp.float32),
            pltpu.SemaphoreType.DMA,
            pltpu.SemaphoreType.DMA,
        ],
    )(d2, idx, tok, d2sc, idxsc, values, eps2d, y)


def kernel(x, W, b, keys, values, epsilons, key_id):
    tok = jnp.minimum(jnp.asarray(key_id, jnp.int32), x.shape[1] - 1)
    x2d = x[0]
    query = lax.dynamic_slice_in_dim(x2d, tok, 1, axis=0)
    eps2d = jnp.pad(epsilons.reshape(_K), (0, _ER * 128 - _K)).reshape(
        _ER, 128)
    d2sc, idxsc = _sc_scan(keys, query.reshape(_D))
    d2, idx, y = _tc_scan_mm(query, keys, x2d, W, b.reshape(1, _D))
    out = _finalize(y, values, eps2d, d2, idx, tok.reshape(1, 1),
                    d2sc, idxsc)
    return out[None]


# speedup vs baseline: 1.0447x; 1.0173x over previous
import functools

import jax
import jax.numpy as jnp
from jax import lax
from jax.experimental import pallas as pl
from jax.experimental.pallas import tpu as pltpu
from jax.experimental.pallas import tpu_sc as plsc

_K = 100000
_D = 768
_S = 2048
_SBLK = 256

_NW = 32
_CH = 72
_NCH = 10
_RPW = _CH * _NCH
_KSC = _NW * _RPW
_KTC = _K - _KSC
_KBLK = 7696
_NJ = _D // 16

_BIG = 2**30
_ER = 800

_DNUMS = lax.GatherDimensionNumbers(
    offset_dims=(), collapsed_slice_dims=(0,), start_index_map=(0,))


def _lane_shuffle(s, perm16):
    return lax.gather(s, perm16[:, None], _DNUMS, (1,),
                      mode=lax.GatherScatterMode.PROMISE_IN_BOUNDS)


def _sc_scan_body(keys_hbm, q_hbm, d2s_hbm, idxs_hbm,
                  q_v, buf0, buf1, bd_v, bi_v, sem0, sem1):
    wid = lax.axis_index("s") * 2 + lax.axis_index("c")
    row0 = _KTC + wid * _RPW

    pltpu.sync_copy(q_hbm, q_v)
    pltpu.make_async_copy(keys_hbm.at[pl.ds(row0, _CH)], buf0, sem0).start()
    pltpu.make_async_copy(
        keys_hbm.at[pl.ds(row0 + _CH, _CH)], buf1, sem1).start()

    lanes = lax.iota(jnp.int32, 16)
    perms = [lanes ^ k for k in (1, 2, 4, 8)]
    ones = jnp.ones((16,), jnp.int32)

    def compute_chunk(buf, carry):
        def rowgrp(g, cr):
            bdv, biv, rv = cr
            base = g * 8
            accs = [jnp.zeros((16,), jnp.float32) for _ in range(8)]
            for j in range(_NJ):
                qj = q_v[pl.ds(j * 16, 16)]
                for r8 in range(8):
                    t = buf[base + r8, pl.ds(j * 16, 16)]
                    d = t - qj
                    accs[r8] = accs[r8] + d * d
            for r8 in range(8):
                s = accs[r8]
                for p in perms:
                    s = s + _lane_shuffle(s, p)
                better = s < bdv
                bdv = jnp.where(better, s, bdv)
                biv = jnp.where(better, rv, biv)
                rv = rv + ones
            return bdv, biv, rv
        return lax.fori_loop(0, _CH // 8, rowgrp, carry)

    def step(c, buf, sem, carry):
        pltpu.make_async_copy(keys_hbm.at[pl.ds(row0, _CH)], buf, sem).wait()
        carry = compute_chunk(buf, carry)

        @pl.when(c + 2 < _NCH)
        def _prefetch():
            pltpu.make_async_copy(
                keys_hbm.at[pl.ds(row0 + (c + 2) * _CH, _CH)], buf,
                sem).start()

        return carry

    def pair(p, carry):
        carry = step(2 * p, buf0, sem0, carry)
        carry = step(2 * p + 1, buf1, sem1, carry)
        return carry

    bdv, biv, _ = lax.fori_loop(
        0, _NCH // 2, pair,
        (jnp.full((16,), jnp.inf, jnp.float32),
         jnp.zeros((16,), jnp.int32),
         jnp.zeros((16,), jnp.int32)))

    for t in range(8):
        bd_v[t, :] = bdv
        bi_v[t, :] = biv
    pltpu.sync_copy(bd_v, d2s_hbm.at[pl.ds(wid * 8, 8)])
    pltpu.sync_copy(bi_v, idxs_hbm.at[pl.ds(wid * 8, 8)])


def _sc_scan(keys, q1d):
    mesh = plsc.VectorSubcoreMesh(core_axis_name="c", subcore_axis_name="s")
    fn = pl.kernel(
        _sc_scan_body,
        out_type=[
            jax.ShapeDtypeStruct((_NW * 8, 16), jnp.float32),
            jax.ShapeDtypeStruct((_NW * 8, 16), jnp.int32),
        ],
        mesh=mesh,
        scratch_types=[
            pltpu.VMEM((_D,), jnp.float32),
            pltpu.VMEM((_CH, _D), jnp.float32),
            pltpu.VMEM((_CH, _D), jnp.float32),
            pltpu.VMEM((8, 16), jnp.float32),
            pltpu.VMEM((8, 16), jnp.int32),
            pltpu.SemaphoreType.DMA,
            pltpu.SemaphoreType.DMA,
        ],
    )
    return fn(keys, q1d)


def _scan_body(q_ref, keys_ref, x_ref, w_ref, b_ref,
               d2_out, idx_out, y_out, bestd_ref, besti_ref):
    step = pl.program_id(0)

    @pl.when(step == 0)
    def _init():
        bestd_ref[0] = jnp.float32(jnp.inf)
        besti_ref[0] = jnp.int32(0)

    @pl.when(step < _S // _SBLK)
    def _mm():
        y = lax.dot_general(
            x_ref[...], w_ref[...], (((1,), (1,)), ((), ())),
            preferred_element_type=jnp.float32)
        y_out[...] = y + b_ref[...]

    q = q_ref[...]
    k = keys_ref[...]
    diff = k - q
    d2 = jnp.sum(diff * diff, axis=1, keepdims=True)
    minv = jnp.min(d2)
    iota = lax.broadcasted_iota(jnp.int32, (_KBLK, 1), 0)
    lidx = jnp.min(jnp.where(d2 == minv, iota, _BIG))

    better = minv < bestd_ref[0]
    bestd_ref[0] = jnp.where(better, minv, bestd_ref[0])
    besti_ref[0] = jnp.where(better, step * _KBLK + lidx, besti_ref[0])

    @pl.when(step == pl.num_programs(0) - 1)
    def _fin():
        d2_out[0, 0] = bestd_ref[0]
        idx_out[0, 0] = besti_ref[0]


def _tc_scan_mm(query_2d, keys, x2d, W, b2d):
    grid = (_KTC // _KBLK,)
    nxb = _S // _SBLK
    return pl.pallas_call(
        _scan_body,
        grid=grid,
        in_specs=[
            pl.BlockSpec((1, _D), lambda i: (0, 0)),
            pl.BlockSpec((_KBLK, _D), lambda i: (i, 0)),
            pl.BlockSpec((_SBLK, _D), lambda i: (jnp.minimum(i, nxb - 1), 0)),
            pl.BlockSpec((_D, _D), lambda i: (0, 0)),
            pl.BlockSpec((1, _D), lambda i: (0, 0)),
        ],
        out_specs=[
            pl.BlockSpec(memory_space=pltpu.SMEM),
            pl.BlockSpec(memory_space=pltpu.SMEM),
            pl.BlockSpec((_SBLK, _D), lambda i: (jnp.minimum(i, nxb - 1), 0)),
        ],
        out_shape=[
            jax.ShapeDtypeStruct((1, 1), jnp.float32),
            jax.ShapeDtypeStruct((1, 1), jnp.int32),
            jax.ShapeDtypeStruct((_S, _D), jnp.float32),
        ],
        scratch_shapes=[
            pltpu.SMEM((1,), jnp.float32),
            pltpu.SMEM((1,), jnp.int32),
        ],
    )(query_2d, keys, x2d, W, b2d)


def _final_body(d2_ref, idx_ref, tok_ref, d2sc_ref, idxsc_ref,
                values_ref, eps_ref, y_ref, out_ref,
                val_ref, epsv_ref, ybuf_ref, sem, sem2):
    d2sc = d2sc_ref[...]
    rows8 = lax.broadcasted_iota(jnp.int32, (_NW * 8, 16), 0)
    gidx = _KTC + (rows8 // 8) * _RPW + idxsc_ref[...]
    msc = jnp.min(d2sc)
    isc = jnp.min(jnp.where(d2sc == msc, gidx, _BIG))
    use_sc = msc < d2_ref[0, 0]
    d2min = jnp.where(use_sc, msc, d2_ref[0, 0])
    idx = jnp.where(use_sc, isc, idx_ref[0, 0])

    copy = pltpu.make_async_copy(values_ref.at[pl.ds(idx, 1)], val_ref, sem)
    copy.start()
    copy2 = pltpu.make_async_copy(
        eps_ref.at[pl.ds(idx // 128, 1)], epsv_ref, sem2)
    copy2.start()
    copy.wait()
    copy2.wait()

    laneio = lax.broadcasted_iota(jnp.int32, (1, 128), 1)
    eps = jnp.sum(jnp.where(laneio == idx % 128, epsv_ref[...],
                            jnp.float32(0.0)))
    cond = (eps >= 0.0) & (d2min <= eps * eps)
    tok = tok_ref[0, 0]

    for b in range(_S // _SBLK):
        @pl.when(cond & (b * _SBLK < tok))
        def _rewrite(b=b):
            ld = pltpu.make_async_copy(
                y_ref.at[pl.ds(b * _SBLK, _SBLK)], ybuf_ref, sem)
            ld.start()
            ld.wait()
            rows = b * _SBLK + lax.broadcasted_iota(
                jnp.int32, (_SBLK, 1), 0)
            ybuf_ref[...] = jnp.where(rows < tok, val_ref[...],
                                      ybuf_ref[...])
            st = pltpu.make_async_copy(
                ybuf_ref, out_ref.at[pl.ds(b * _SBLK, _SBLK)], sem)
            st.start()
            st.wait()


def _finalize(y, values, eps2d, d2, idx, tok, d2sc, idxsc):
    return pl.pallas_call(
        _final_body,
        in_specs=[
            pl.BlockSpec(memory_space=pltpu.SMEM),
            pl.BlockSpec(memory_space=pltpu.SMEM),
            pl.BlockSpec(memory_space=pltpu.SMEM),
            pl.BlockSpec((_NW * 8, 16), lambda: (0, 0)),
            pl.BlockSpec((_NW * 8, 16), lambda: (0, 0)),
            pl.BlockSpec(memory_space=pl.ANY),
            pl.BlockSpec(memory_space=pl.ANY),
            pl.BlockSpec(memory_space=pl.ANY),
        ],
        out_specs=pl.BlockSpec(memory_space=pl.ANY),
        out_shape=jax.ShapeDtypeStruct((_S, _D), jnp.float32),
        input_output_aliases={7: 0},
        scratch_shapes=[
            pltpu.VMEM((1, _D), jnp.float32),
            pltpu.VMEM((1, 128), jnp.float32),
            pltpu.VMEM((_SBLK, _D), jnp.float32),
            pltpu.SemaphoreType.DMA,
            pltpu.SemaphoreType.DMA,
        ],
    )(d2, idx, tok, d2sc, idxsc, values, eps2d, y)


def kernel(x, W, b, keys, values, epsilons, key_id):
    tok = jnp.minimum(jnp.asarray(key_id, jnp.int32), x.shape[1] - 1)
    x2d = x[0]
    query = lax.dynamic_slice_in_dim(x2d, tok, 1, axis=0)
    eps2d = jnp.pad(epsilons.reshape(_K), (0, _ER * 128 - _K)).reshape(
        _ER, 128)
    d2sc, idxsc = _sc_scan(keys, query.reshape(_D))
    d2, idx, y = _tc_scan_mm(query, keys, x2d, W, b.reshape(1, _D))
    out = _finalize(y, values, eps2d, d2, idx, tok.reshape(1, 1),
                    d2sc, idxsc)
    return out[None]


# speedup vs baseline: 1.0594x; 1.0141x over previous
import functools

import jax
import jax.numpy as jnp
from jax import lax
from jax.experimental import pallas as pl
from jax.experimental.pallas import tpu as pltpu
from jax.experimental.pallas import tpu_sc as plsc

_K = 100000
_D = 768
_S = 2048
_SBLK = 256

_NW = 32
_CH = 72
_NCH = 10
_RPW = _CH * _NCH
_KSC = _NW * _RPW
_KTC = _K - _KSC
_KBLK = 3848
_NJ = _D // 16

_BIG = 2**30
_ER = 800

_DNUMS = lax.GatherDimensionNumbers(
    offset_dims=(), collapsed_slice_dims=(0,), start_index_map=(0,))


def _lane_shuffle(s, perm16):
    return lax.gather(s, perm16[:, None], _DNUMS, (1,),
                      mode=lax.GatherScatterMode.PROMISE_IN_BOUNDS)


def _sc_scan_body(keys_hbm, q_hbm, d2s_hbm, idxs_hbm,
                  q_v, buf0, buf1, bd_v, bi_v, sem0, sem1):
    wid = lax.axis_index("s") * 2 + lax.axis_index("c")
    row0 = _KTC + wid * _RPW

    pltpu.sync_copy(q_hbm, q_v)
    pltpu.make_async_copy(keys_hbm.at[pl.ds(row0, _CH)], buf0, sem0).start()
    pltpu.make_async_copy(
        keys_hbm.at[pl.ds(row0 + _CH, _CH)], buf1, sem1).start()

    lanes = lax.iota(jnp.int32, 16)
    perms = [lanes ^ k for k in (1, 2, 4, 8)]
    ones = jnp.ones((16,), jnp.int32)

    def compute_chunk(buf, carry):
        def rowgrp(g, cr):
            bdv, biv, rv = cr
            base = g * 8
            accs = [jnp.zeros((16,), jnp.float32) for _ in range(8)]
            for j in range(_NJ):
                qj = q_v[pl.ds(j * 16, 16)]
                for r8 in range(8):
                    t = buf[base + r8, pl.ds(j * 16, 16)]
                    d = t - qj
                    accs[r8] = accs[r8] + d * d
            for r8 in range(8):
                s = accs[r8]
                for p in perms:
                    s = s + _lane_shuffle(s, p)
                better = s < bdv
                bdv = jnp.where(better, s, bdv)
                biv = jnp.where(better, rv, biv)
                rv = rv + ones
            return bdv, biv, rv
        return lax.fori_loop(0, _CH // 8, rowgrp, carry)

    def step(c, buf, sem, carry):
        pltpu.make_async_copy(keys_hbm.at[pl.ds(row0, _CH)], buf, sem).wait()
        carry = compute_chunk(buf, carry)

        @pl.when(c + 2 < _NCH)
        def _prefetch():
            pltpu.make_async_copy(
                keys_hbm.at[pl.ds(row0 + (c + 2) * _CH, _CH)], buf,
                sem).start()

        return carry

    def pair(p, carry):
        carry = step(2 * p, buf0, sem0, carry)
        carry = step(2 * p + 1, buf1, sem1, carry)
        return carry

    bdv, biv, _ = lax.fori_loop(
        0, _NCH // 2, pair,
        (jnp.full((16,), jnp.inf, jnp.float32),
         jnp.zeros((16,), jnp.int32),
         jnp.zeros((16,), jnp.int32)))

    for t in range(8):
        bd_v[t, :] = bdv
        bi_v[t, :] = biv
    pltpu.sync_copy(bd_v, d2s_hbm.at[pl.ds(wid * 8, 8)])
    pltpu.sync_copy(bi_v, idxs_hbm.at[pl.ds(wid * 8, 8)])


def _sc_scan(keys, q1d):
    mesh = plsc.VectorSubcoreMesh(core_axis_name="c", subcore_axis_name="s")
    fn = pl.kernel(
        _sc_scan_body,
        out_type=[
            jax.ShapeDtypeStruct((_NW * 8, 16), jnp.float32),
            jax.ShapeDtypeStruct((_NW * 8, 16), jnp.int32),
        ],
        mesh=mesh,
        scratch_types=[
            pltpu.VMEM((_D,), jnp.float32),
            pltpu.VMEM((_CH, _D), jnp.float32),
            pltpu.VMEM((_CH, _D), jnp.float32),
            pltpu.VMEM((8, 16), jnp.float32),
            pltpu.VMEM((8, 16), jnp.int32),
            pltpu.SemaphoreType.DMA,
            pltpu.SemaphoreType.DMA,
        ],
    )
    return fn(keys, q1d)


def _scan_body(q_ref, keys_ref, x_ref, w_ref, b_ref,
               d2_out, idx_out, y_out, bestd_ref, besti_ref):
    step = pl.program_id(0)

    @pl.when(step == 0)
    def _init():
        bestd_ref[0] = jnp.float32(jnp.inf)
        besti_ref[0] = jnp.int32(0)

    @pl.when(step < _S // _SBLK)
    def _mm():
        y = lax.dot_general(
            x_ref[...], w_ref[...], (((1,), (1,)), ((), ())),
            preferred_element_type=jnp.float32)
        y_out[...] = y + b_ref[...]

    q = q_ref[...]
    k = keys_ref[...]
    diff = k - q
    d2 = jnp.sum(diff * diff, axis=1, keepdims=True)
    minv = jnp.min(d2)
    iota = lax.broadcasted_iota(jnp.int32, (_KBLK, 1), 0)
    lidx = jnp.min(jnp.where(d2 == minv, iota, _BIG))

    better = minv < bestd_ref[0]
    bestd_ref[0] = jnp.where(better, minv, bestd_ref[0])
    besti_ref[0] = jnp.where(better, step * _KBLK + lidx, besti_ref[0])

    @pl.when(step == pl.num_programs(0) - 1)
    def _fin():
        d2_out[0, 0] = bestd_ref[0]
        idx_out[0, 0] = besti_ref[0]


def _tc_scan_mm(query_2d, keys, x2d, W, b2d):
    grid = (_KTC // _KBLK,)
    nxb = _S // _SBLK
    return pl.pallas_call(
        _scan_body,
        grid=grid,
        in_specs=[
            pl.BlockSpec((1, _D), lambda i: (0, 0)),
            pl.BlockSpec((_KBLK, _D), lambda i: (i, 0)),
            pl.BlockSpec((_SBLK, _D), lambda i: (jnp.minimum(i, nxb - 1), 0)),
            pl.BlockSpec((_D, _D), lambda i: (0, 0)),
            pl.BlockSpec((1, _D), lambda i: (0, 0)),
        ],
        out_specs=[
            pl.BlockSpec(memory_space=pltpu.SMEM),
            pl.BlockSpec(memory_space=pltpu.SMEM),
            pl.BlockSpec((_SBLK, _D), lambda i: (jnp.minimum(i, nxb - 1), 0)),
        ],
        out_shape=[
            jax.ShapeDtypeStruct((1, 1), jnp.float32),
            jax.ShapeDtypeStruct((1, 1), jnp.int32),
            jax.ShapeDtypeStruct((_S, _D), jnp.float32),
        ],
        scratch_shapes=[
            pltpu.SMEM((1,), jnp.float32),
            pltpu.SMEM((1,), jnp.int32),
        ],
    )(query_2d, keys, x2d, W, b2d)


def _final_body(d2_ref, idx_ref, tok_ref, d2sc_ref, idxsc_ref,
                values_ref, eps_ref, y_ref, out_ref,
                val_ref, epsv_ref, ybuf_ref, sem, sem2):
    d2sc = d2sc_ref[...]
    rows8 = lax.broadcasted_iota(jnp.int32, (_NW * 8, 16), 0)
    gidx = _KTC + (rows8 // 8) * _RPW + idxsc_ref[...]
    msc = jnp.min(d2sc)
    isc = jnp.min(jnp.where(d2sc == msc, gidx, _BIG))
    use_sc = msc < d2_ref[0, 0]
    d2min = jnp.where(use_sc, msc, d2_ref[0, 0])
    idx = jnp.where(use_sc, isc, idx_ref[0, 0])

    copy = pltpu.make_async_copy(values_ref.at[pl.ds(idx, 1)], val_ref, sem)
    copy.start()
    copy2 = pltpu.make_async_copy(
        eps_ref.at[pl.ds(idx // 128, 1)], epsv_ref, sem2)
    copy2.start()
    copy.wait()
    copy2.wait()

    laneio = lax.broadcasted_iota(jnp.int32, (1, 128), 1)
    eps = jnp.sum(jnp.where(laneio == idx % 128, epsv_ref[...],
                            jnp.float32(0.0)))
    cond = (eps >= 0.0) & (d2min <= eps * eps)
    tok = tok_ref[0, 0]

    for b in range(_S // _SBLK):
        @pl.when(cond & (b * _SBLK < tok))
        def _rewrite(b=b):
            ld = pltpu.make_async_copy(
                y_ref.at[pl.ds(b * _SBLK, _SBLK)], ybuf_ref, sem)
            ld.start()
            ld.wait()
            rows = b * _SBLK + lax.broadcasted_iota(
                jnp.int32, (_SBLK, 1), 0)
            ybuf_ref[...] = jnp.where(rows < tok, val_ref[...],
                                      ybuf_ref[...])
            st = pltpu.make_async_copy(
                ybuf_ref, out_ref.at[pl.ds(b * _SBLK, _SBLK)], sem)
            st.start()
            st.wait()


def _finalize(y, values, eps2d, d2, idx, tok, d2sc, idxsc):
    return pl.pallas_call(
        _final_body,
        in_specs=[
            pl.BlockSpec(memory_space=pltpu.SMEM),
            pl.BlockSpec(memory_space=pltpu.SMEM),
            pl.BlockSpec(memory_space=pltpu.SMEM),
            pl.BlockSpec((_NW * 8, 16), lambda: (0, 0)),
            pl.BlockSpec((_NW * 8, 16), lambda: (0, 0)),
            pl.BlockSpec(memory_space=pl.ANY),
            pl.BlockSpec(memory_space=pl.ANY),
            pl.BlockSpec(memory_space=pl.ANY),
        ],
        out_specs=pl.BlockSpec(memory_space=pl.ANY),
        out_shape=jax.ShapeDtypeStruct((_S, _D), jnp.float32),
        input_output_aliases={7: 0},
        scratch_shapes=[
            pltpu.VMEM((1, _D), jnp.float32),
            pltpu.VMEM((1, 128), jnp.float32),
            pltpu.VMEM((_SBLK, _D), jnp.float32),
            pltpu.SemaphoreType.DMA,
            pltpu.SemaphoreType.DMA,
        ],
    )(d2, idx, tok, d2sc, idxsc, values, eps2d, y)


def kernel(x, W, b, keys, values, epsilons, key_id):
    tok = jnp.minimum(jnp.asarray(key_id, jnp.int32), x.shape[1] - 1)
    x2d = x[0]
    query = lax.dynamic_slice_in_dim(x2d, tok, 1, axis=0)
    eps2d = jnp.pad(epsilons.reshape(_K), (0, _ER * 128 - _K)).reshape(
        _ER, 128)
    d2sc, idxsc = _sc_scan(keys, query.reshape(_D))
    d2, idx, y = _tc_scan_mm(query, keys, x2d, W, b.reshape(1, _D))
    out = _finalize(y, values, eps2d, d2, idx, tok.reshape(1, 1),
                    d2sc, idxsc)
    return out[None]


# speedup vs baseline: 1.0950x; 1.0337x over previous
import functools

import jax
import jax.numpy as jnp
from jax import lax
from jax.experimental import pallas as pl
from jax.experimental.pallas import tpu as pltpu
from jax.experimental.pallas import tpu_sc as plsc

_K = 100000
_D = 768
_S = 2048
_SBLK = 256

_NW = 32
_CH = 48
_NCH = 10
_RPW = _CH * _NCH
_KSC = _NW * _RPW
_KTC = _K - _KSC
_KBLK = 4232
_NJ = _D // 16

_BIG = 2**30
_ER = 800

_DNUMS = lax.GatherDimensionNumbers(
    offset_dims=(), collapsed_slice_dims=(0,), start_index_map=(0,))


def _lane_shuffle(s, perm16):
    return lax.gather(s, perm16[:, None], _DNUMS, (1,),
                      mode=lax.GatherScatterMode.PROMISE_IN_BOUNDS)


def _sc_scan_body(keys_hbm, q_hbm, d2s_hbm, idxs_hbm,
                  q_v, buf0, buf1, bd_v, bi_v, sem0, sem1):
    wid = lax.axis_index("s") * 2 + lax.axis_index("c")
    row0 = _KTC + wid * _RPW

    pltpu.sync_copy(q_hbm, q_v)
    pltpu.make_async_copy(keys_hbm.at[pl.ds(row0, _CH)], buf0, sem0).start()
    pltpu.make_async_copy(
        keys_hbm.at[pl.ds(row0 + _CH, _CH)], buf1, sem1).start()

    lanes = lax.iota(jnp.int32, 16)
    perms = [lanes ^ k for k in (1, 2, 4, 8)]
    ones = jnp.ones((16,), jnp.int32)

    def compute_chunk(buf, carry):
        def rowgrp(g, cr):
            bdv, biv, rv = cr
            base = g * 8
            accs = [jnp.zeros((16,), jnp.float32) for _ in range(8)]
            for j in range(_NJ):
                qj = q_v[pl.ds(j * 16, 16)]
                for r8 in range(8):
                    t = buf[base + r8, pl.ds(j * 16, 16)]
                    d = t - qj
                    accs[r8] = accs[r8] + d * d
            for r8 in range(8):
                s = accs[r8]
                for p in perms:
                    s = s + _lane_shuffle(s, p)
                better = s < bdv
                bdv = jnp.where(better, s, bdv)
                biv = jnp.where(better, rv, biv)
                rv = rv + ones
            return bdv, biv, rv
        return lax.fori_loop(0, _CH // 8, rowgrp, carry)

    def step(c, buf, sem, carry):
        pltpu.make_async_copy(keys_hbm.at[pl.ds(row0, _CH)], buf, sem).wait()
        carry = compute_chunk(buf, carry)

        @pl.when(c + 2 < _NCH)
        def _prefetch():
            pltpu.make_async_copy(
                keys_hbm.at[pl.ds(row0 + (c + 2) * _CH, _CH)], buf,
                sem).start()

        return carry

    def pair(p, carry):
        carry = step(2 * p, buf0, sem0, carry)
        carry = step(2 * p + 1, buf1, sem1, carry)
        return carry

    bdv, biv, _ = lax.fori_loop(
        0, _NCH // 2, pair,
        (jnp.full((16,), jnp.inf, jnp.float32),
         jnp.zeros((16,), jnp.int32),
         jnp.zeros((16,), jnp.int32)))

    for t in range(8):
        bd_v[t, :] = bdv
        bi_v[t, :] = biv
    pltpu.sync_copy(bd_v, d2s_hbm.at[pl.ds(wid * 8, 8)])
    pltpu.sync_copy(bi_v, idxs_hbm.at[pl.ds(wid * 8, 8)])


def _sc_scan(keys, q1d):
    mesh = plsc.VectorSubcoreMesh(core_axis_name="c", subcore_axis_name="s")
    fn = pl.kernel(
        _sc_scan_body,
        out_type=[
            jax.ShapeDtypeStruct((_NW * 8, 16), jnp.float32),
            jax.ShapeDtypeStruct((_NW * 8, 16), jnp.int32),
        ],
        mesh=mesh,
        scratch_types=[
            pltpu.VMEM((_D,), jnp.float32),
            pltpu.VMEM((_CH, _D), jnp.float32),
            pltpu.VMEM((_CH, _D), jnp.float32),
            pltpu.VMEM((8, 16), jnp.float32),
            pltpu.VMEM((8, 16), jnp.int32),
            pltpu.SemaphoreType.DMA,
            pltpu.SemaphoreType.DMA,
        ],
    )
    return fn(keys, q1d)


def _scan_body(q_ref, keys_ref, x_ref, w_ref, b_ref,
               d2_out, idx_out, y_out, bestd_ref, besti_ref):
    step = pl.program_id(0)

    @pl.when(step == 0)
    def _init():
        bestd_ref[0] = jnp.float32(jnp.inf)
        besti_ref[0] = jnp.int32(0)

    @pl.when(step < _S // _SBLK)
    def _mm():
        y = lax.dot_general(
            x_ref[...], w_ref[...], (((1,), (1,)), ((), ())),
            preferred_element_type=jnp.float32)
        y_out[...] = y + b_ref[...]

    q = q_ref[...]
    k = keys_ref[...]
    diff = k - q
    d2 = jnp.sum(diff * diff, axis=1, keepdims=True)
    minv = jnp.min(d2)
    iota = lax.broadcasted_iota(jnp.int32, (_KBLK, 1), 0)
    lidx = jnp.min(jnp.where(d2 == minv, iota, _BIG))

    better = minv < bestd_ref[0]
    bestd_ref[0] = jnp.where(better, minv, bestd_ref[0])
    besti_ref[0] = jnp.where(better, step * _KBLK + lidx, besti_ref[0])

    @pl.when(step == pl.num_programs(0) - 1)
    def _fin():
        d2_out[0, 0] = bestd_ref[0]
        idx_out[0, 0] = besti_ref[0]


def _tc_scan_mm(query_2d, keys, x2d, W, b2d):
    grid = (_KTC // _KBLK,)
    nxb = _S // _SBLK
    return pl.pallas_call(
        _scan_body,
        grid=grid,
        in_specs=[
            pl.BlockSpec((1, _D), lambda i: (0, 0)),
            pl.BlockSpec((_KBLK, _D), lambda i: (i, 0)),
            pl.BlockSpec((_SBLK, _D), lambda i: (jnp.minimum(i, nxb - 1), 0)),
            pl.BlockSpec((_D, _D), lambda i: (0, 0)),
            pl.BlockSpec((1, _D), lambda i: (0, 0)),
        ],
        out_specs=[
            pl.BlockSpec(memory_space=pltpu.SMEM),
            pl.BlockSpec(memory_space=pltpu.SMEM),
            pl.BlockSpec((_SBLK, _D), lambda i: (jnp.minimum(i, nxb - 1), 0)),
        ],
        out_shape=[
            jax.ShapeDtypeStruct((1, 1), jnp.float32),
            jax.ShapeDtypeStruct((1, 1), jnp.int32),
            jax.ShapeDtypeStruct((_S, _D), jnp.float32),
        ],
        scratch_shapes=[
            pltpu.SMEM((1,), jnp.float32),
            pltpu.SMEM((1,), jnp.int32),
        ],
    )(query_2d, keys, x2d, W, b2d)


def _final_body(d2_ref, idx_ref, tok_ref, d2sc_ref, idxsc_ref,
                values_ref, eps_ref, y_ref, out_ref,
                val_ref, epsv_ref, ybuf_ref, sem, sem2):
    d2sc = d2sc_ref[...]
    rows8 = lax.broadcasted_iota(jnp.int32, (_NW * 8, 16), 0)
    gidx = _KTC + (rows8 // 8) * _RPW + idxsc_ref[...]
    msc = jnp.min(d2sc)
    isc = jnp.min(jnp.where(d2sc == msc, gidx, _BIG))
    use_sc = msc < d2_ref[0, 0]
    d2min = jnp.where(use_sc, msc, d2_ref[0, 0])
    idx = jnp.where(use_sc, isc, idx_ref[0, 0])

    copy = pltpu.make_async_copy(values_ref.at[pl.ds(idx, 1)], val_ref, sem)
    copy.start()
    copy2 = pltpu.make_async_copy(
        eps_ref.at[pl.ds(idx // 128, 1)], epsv_ref, sem2)
    copy2.start()
    copy.wait()
    copy2.wait()

    laneio = lax.broadcasted_iota(jnp.int32, (1, 128), 1)
    eps = jnp.sum(jnp.where(laneio == idx % 128, epsv_ref[...],
                            jnp.float32(0.0)))
    cond = (eps >= 0.0) & (d2min <= eps * eps)
    tok = tok_ref[0, 0]

    for b in range(_S // _SBLK):
        @pl.when(cond & (b * _SBLK < tok))
        def _rewrite(b=b):
            ld = pltpu.make_async_copy(
                y_ref.at[pl.ds(b * _SBLK, _SBLK)], ybuf_ref, sem)
            ld.start()
            ld.wait()
            rows = b * _SBLK + lax.broadcasted_iota(
                jnp.int32, (_SBLK, 1), 0)
            ybuf_ref[...] = jnp.where(rows < tok, val_ref[...],
                                      ybuf_ref[...])
            st = pltpu.make_async_copy(
                ybuf_ref, out_ref.at[pl.ds(b * _SBLK, _SBLK)], sem)
            st.start()
            st.wait()


def _finalize(y, values, eps2d, d2, idx, tok, d2sc, idxsc):
    return pl.pallas_call(
        _final_body,
        in_specs=[
            pl.BlockSpec(memory_space=pltpu.SMEM),
            pl.BlockSpec(memory_space=pltpu.SMEM),
            pl.BlockSpec(memory_space=pltpu.SMEM),
            pl.BlockSpec((_NW * 8, 16), lambda: (0, 0)),
            pl.BlockSpec((_NW * 8, 16), lambda: (0, 0)),
            pl.BlockSpec(memory_space=pl.ANY),
            pl.BlockSpec(memory_space=pl.ANY),
            pl.BlockSpec(memory_space=pl.ANY),
        ],
        out_specs=pl.BlockSpec(memory_space=pl.ANY),
        out_shape=jax.ShapeDtypeStruct((_S, _D), jnp.float32),
        input_output_aliases={7: 0},
        scratch_shapes=[
            pltpu.VMEM((1, _D), jnp.float32),
            pltpu.VMEM((1, 128), jnp.float32),
            pltpu.VMEM((_SBLK, _D), jnp.float32),
            pltpu.SemaphoreType.DMA,
            pltpu.SemaphoreType.DMA,
        ],
    )(d2, idx, tok, d2sc, idxsc, values, eps2d, y)


def kernel(x, W, b, keys, values, epsilons, key_id):
    tok = jnp.minimum(jnp.asarray(key_id, jnp.int32), x.shape[1] - 1)
    x2d = x[0]
    query = lax.dynamic_slice_in_dim(x2d, tok, 1, axis=0)
    eps2d = jnp.pad(epsilons.reshape(_K), (0, _ER * 128 - _K)).reshape(
        _ER, 128)
    d2sc, idxsc = _sc_scan(keys, query.reshape(_D))
    d2, idx, y = _tc_scan_mm(query, keys, x2d, W, b.reshape(1, _D))
    out = _finalize(y, values, eps2d, d2, idx, tok.reshape(1, 1),
                    d2sc, idxsc)
    return out[None]
